# Initial kernel scaffold; baseline (speedup 1.0000x reference)
import functools
import jax
import jax.numpy as jnp
from jax import lax
from jax.experimental import pallas as pl
from jax.experimental.pallas import tpu as pltpu, tpu_sc as plsc

_B, _S = 4, 4096
_D = 1024
_L = 16
_NW = 32
_ROWS_PER_W = (_B * _S) // _NW
_CHUNKS_PER_ROW = _S // _ROWS_PER_W
_GCHUNK = 32
_NG = _ROWS_PER_W // _GCHUNK


def _body(inp_hbm, table_hbm, out_hbm, inp_v, idx_v, rows_v, sem):
    nc = 2
    wid = lax.axis_index("s") * nc + lax.axis_index("c")
    b = wid // _CHUNKS_PER_ROW
    c = wid % _CHUNKS_PER_ROW

    pltpu.sync_copy(inp_hbm.at[b], inp_v)

    def count_step(j, carry):
        x = inp_v[pl.ds(j * _L, _L)]
        m = jnp.where(x != 0, 1, 0).astype(jnp.int32)
        return carry + jnp.sum(m)

    nprefix_vecs = c * (_ROWS_PER_W // _L)
    carry0 = lax.fori_loop(0, nprefix_vecs, count_step, jnp.int32(0))

    base_vec = nprefix_vecs

    def pos_step(j, carry):
        x = inp_v[pl.ds((base_vec + j) * _L, _L)]
        m = jnp.where(x != 0, 1, 0).astype(jnp.int32)
        pos = (carry + jnp.cumsum(m)) * m
        idx_v[pl.ds(j * _L, _L)] = pos
        return carry + jnp.sum(m)

    lax.fori_loop(0, _ROWS_PER_W // _L, pos_step, carry0)

    out_base = wid * _ROWS_PER_W
    for g in range(_NG):
        pltpu.async_copy(
            table_hbm.at[idx_v.at[pl.ds(g * _GCHUNK, _GCHUNK)]], rows_v, sem
        ).wait()
        pltpu.sync_copy(rows_v, out_hbm.at[pl.ds(out_base + g * _GCHUNK, _GCHUNK)])


@jax.jit
def _run(inp, weights):
    mesh = plsc.VectorSubcoreMesh(core_axis_name="c", subcore_axis_name="s")
    k = functools.partial(
        pl.kernel,
        mesh=mesh,
        out_type=jax.ShapeDtypeStruct((_B * _S, _D), jnp.float32),
        scratch_types=[
            pltpu.VMEM((_S,), jnp.int32),
            pltpu.VMEM((_ROWS_PER_W,), jnp.int32),
            pltpu.VMEM((_GCHUNK, _D), jnp.float32),
            pltpu.SemaphoreType.DMA,
        ],
    )(_body)
    return k(inp, weights)


def kernel(input, weights):
    out = _run(input, weights)
    return lax.stop_gradient(out.reshape(_B, _S, _D))


# speedup vs baseline: 1.6385x; 1.6385x over previous
import functools
import jax
import jax.numpy as jnp
from jax import lax
from jax.experimental import pallas as pl
from jax.experimental.pallas import tpu as pltpu, tpu_sc as plsc

_B, _S = 4, 4096
_D = 1024
_L = 16
_NW = 32
_ROWS_PER_W = (_B * _S) // _NW
_CHUNKS_PER_ROW = _S // _ROWS_PER_W
_GCHUNK = 32
_NG = _ROWS_PER_W // _GCHUNK


def _body(inp_hbm, table_hbm, out_hbm, inp_v, idx_v, rows_v, sem):
    nc = 2
    wid = lax.axis_index("s") * nc + lax.axis_index("c")
    b = wid // _CHUNKS_PER_ROW
    c = wid % _CHUNKS_PER_ROW

    pltpu.sync_copy(inp_hbm.at[b], inp_v)

    def count_step(j, carry):
        x = inp_v[pl.ds(j * _L, _L)]
        m = jnp.where(x != 0, 1, 0).astype(jnp.int32)
        return carry + jnp.sum(m)

    nprefix_vecs = c * (_ROWS_PER_W // _L)
    carry0 = lax.fori_loop(0, nprefix_vecs, count_step, jnp.int32(0))

    base_vec = nprefix_vecs

    def pos_step(j, carry):
        x = inp_v[pl.ds((base_vec + j) * _L, _L)]
        m = jnp.where(x != 0, 1, 0).astype(jnp.int32)
        pos = (carry + jnp.cumsum(m)) * m
        idx_v[pl.ds(j * _L, _L)] = pos
        return carry + jnp.sum(m)

    lax.fori_loop(0, _ROWS_PER_W // _L, pos_step, carry0)

    out_base = wid * _ROWS_PER_W
    for g in range(_NG):
        pltpu.async_copy(
            table_hbm.at[idx_v.at[pl.ds(g * _GCHUNK, _GCHUNK)]], rows_v, sem
        ).wait()
        pltpu.sync_copy(rows_v, out_hbm.at[pl.ds(out_base + g * _GCHUNK, _GCHUNK)])


@jax.jit
def _run(inp, weights):
    mesh = plsc.VectorSubcoreMesh(core_axis_name="c", subcore_axis_name="s")
    k = functools.partial(
        pl.kernel,
        mesh=mesh,
        out_type=jax.ShapeDtypeStruct((_B * _S, _D), jnp.float32),
        scratch_types=[
            pltpu.VMEM((_S,), jnp.int32),
            pltpu.VMEM((_ROWS_PER_W,), jnp.int32),
            pltpu.VMEM((_GCHUNK, _D), jnp.float32),
            pltpu.SemaphoreType.DMA,
        ],
        compiler_params=pltpu.CompilerParams(needs_layout_passes=False),
    )(_body)
    return k(inp, weights)


def kernel(input, weights):
    out = _run(input, weights)
    return lax.stop_gradient(out.reshape(_B, _S, _D))


# speedup vs baseline: 1.8418x; 1.1241x over previous
import functools
import jax
import jax.numpy as jnp
from jax import lax
from jax.experimental import pallas as pl
from jax.experimental.pallas import tpu as pltpu, tpu_sc as plsc

_B, _S = 4, 4096
_D = 1024
_L = 16
_NW = 32
_ROWS_PER_W = (_B * _S) // _NW
_CHUNKS_PER_ROW = _S // _ROWS_PER_W
_GCHUNK = 32
_NG = _ROWS_PER_W // _GCHUNK


def _body(
    inp_hbm, table_hbm, out_hbm, inp_v, idx_v, rows0, rows1, gsem, wsem0, wsem1
):
    nc = 2
    wid = lax.axis_index("s") * nc + lax.axis_index("c")
    b = wid // _CHUNKS_PER_ROW
    c = wid % _CHUNKS_PER_ROW
    rows = (rows0, rows1)
    wsem = (wsem0, wsem1)

    pltpu.sync_copy(inp_hbm.at[b], inp_v)

    def count_step(j, acc):
        x = inp_v[pl.ds(j * _L, _L)]
        return acc + jnp.where(x != 0, 1, 0).astype(jnp.int32)

    nprefix_vecs = c * (_ROWS_PER_W // _L)
    acc = lax.fori_loop(0, nprefix_vecs, count_step, jnp.zeros((_L,), jnp.int32))
    carry0 = jnp.sum(acc)

    base_vec = nprefix_vecs

    def pos_step(j, carry):
        x = inp_v[pl.ds((base_vec + j) * _L, _L)]
        m = jnp.where(x != 0, 1, 0).astype(jnp.int32)
        pos = (carry + jnp.cumsum(m)) * m
        idx_v[pl.ds(j * _L, _L)] = pos
        return carry + jnp.sum(m)

    lax.fori_loop(0, _ROWS_PER_W // _L, pos_step, carry0)

    out_base = wid * _ROWS_PER_W

    def gather(g, p):
        return pltpu.async_copy(
            table_hbm.at[idx_v.at[pl.ds(g * _GCHUNK, _GCHUNK)]], rows[p], gsem
        )

    writes = [None] * _NG
    gathers = [None] * _NG
    gathers[0] = gather(0, 0)
    for g in range(_NG):
        p = g & 1
        gathers[g].wait()
        if g + 1 < _NG:
            if g >= 1:
                writes[g - 1].wait()
            gathers[g + 1] = gather(g + 1, (g + 1) & 1)
        writes[g] = pltpu.async_copy(
            rows[p], out_hbm.at[pl.ds(out_base + g * _GCHUNK, _GCHUNK)], wsem[p]
        )
    writes[_NG - 2].wait()
    writes[_NG - 1].wait()


@jax.jit
def _run(inp, weights):
    mesh = plsc.VectorSubcoreMesh(core_axis_name="c", subcore_axis_name="s")
    k = functools.partial(
        pl.kernel,
        mesh=mesh,
        out_type=jax.ShapeDtypeStruct((_B * _S, _D), jnp.float32),
        scratch_types=[
            pltpu.VMEM((_S,), jnp.int32),
            pltpu.VMEM((_ROWS_PER_W,), jnp.int32),
            pltpu.VMEM((_GCHUNK, _D), jnp.float32),
            pltpu.VMEM((_GCHUNK, _D), jnp.float32),
            pltpu.SemaphoreType.DMA,
            pltpu.SemaphoreType.DMA,
            pltpu.SemaphoreType.DMA,
        ],
        compiler_params=pltpu.CompilerParams(needs_layout_passes=False),
    )(_body)
    return k(inp, weights)


def kernel(input, weights):
    out = _run(input, weights)
    return lax.stop_gradient(out.reshape(_B, _S, _D))


# speedup vs baseline: 1.9206x; 1.0428x over previous
import functools
import jax
import jax.numpy as jnp
from jax import lax
from jax.experimental import pallas as pl
from jax.experimental.pallas import tpu as pltpu, tpu_sc as plsc

_B, _S = 4, 4096
_D = 1024
_L = 16
_NW = 32
_ROWS_PER_W = (_B * _S) // _NW
_CHUNKS_PER_ROW = _S // _ROWS_PER_W
_GCHUNK = 32
_NG = _ROWS_PER_W // _GCHUNK


def _body(
    inp_hbm,
    table_hbm,
    out_hbm,
    inp_v,
    idx_v,
    rows0,
    rows1,
    rows2,
    gsem0,
    gsem1,
    wsem0,
    wsem1,
    wsem2,
):
    nc = 2
    wid = lax.axis_index("s") * nc + lax.axis_index("c")
    b = wid // _CHUNKS_PER_ROW
    c = wid % _CHUNKS_PER_ROW
    rows = (rows0, rows1, rows2)
    gsem = (gsem0, gsem1)
    wsem = (wsem0, wsem1, wsem2)

    pltpu.sync_copy(inp_hbm.at[b], inp_v)

    def count_step(j, acc):
        x = inp_v[pl.ds(j * _L, _L)]
        return acc + jnp.where(x != 0, 1, 0).astype(jnp.int32)

    nprefix_vecs = c * (_ROWS_PER_W // _L)
    acc = lax.fori_loop(0, nprefix_vecs, count_step, jnp.zeros((_L,), jnp.int32))
    carry0 = jnp.sum(acc)

    base_vec = nprefix_vecs

    def pos_step(j, carry):
        x = inp_v[pl.ds((base_vec + j) * _L, _L)]
        m = jnp.where(x != 0, 1, 0).astype(jnp.int32)
        pos = (carry + jnp.cumsum(m)) * m
        idx_v[pl.ds(j * _L, _L)] = pos
        return carry + jnp.sum(m)

    lax.fori_loop(0, _ROWS_PER_W // _L, pos_step, carry0)

    out_base = wid * _ROWS_PER_W

    def gather(g):
        return pltpu.async_copy(
            table_hbm.at[idx_v.at[pl.ds(g * _GCHUNK, _GCHUNK)]],
            rows[g % 3],
            gsem[g % 2],
        )

    writes = [None] * _NG
    gathers = [None] * _NG
    gathers[0] = gather(0)
    gathers[1] = gather(1)
    for g in range(_NG):
        p = g % 3
        gathers[g].wait()
        if g + 2 < _NG:
            if g >= 1:
                writes[g - 1].wait()
            gathers[g + 2] = gather(g + 2)
        writes[g] = pltpu.async_copy(
            rows[p], out_hbm.at[pl.ds(out_base + g * _GCHUNK, _GCHUNK)], wsem[p]
        )
    writes[_NG - 3].wait()
    writes[_NG - 2].wait()
    writes[_NG - 1].wait()


@jax.jit
def _run(inp, weights):
    mesh = plsc.VectorSubcoreMesh(core_axis_name="c", subcore_axis_name="s")
    k = functools.partial(
        pl.kernel,
        mesh=mesh,
        out_type=jax.ShapeDtypeStruct((_B * _S, _D), jnp.float32),
        scratch_types=[
            pltpu.VMEM((_S,), jnp.int32),
            pltpu.VMEM((_ROWS_PER_W,), jnp.int32),
            pltpu.VMEM((_GCHUNK, _D), jnp.float32),
            pltpu.VMEM((_GCHUNK, _D), jnp.float32),
            pltpu.VMEM((_GCHUNK, _D), jnp.float32),
            pltpu.SemaphoreType.DMA,
            pltpu.SemaphoreType.DMA,
            pltpu.SemaphoreType.DMA,
            pltpu.SemaphoreType.DMA,
            pltpu.SemaphoreType.DMA,
        ],
        compiler_params=pltpu.CompilerParams(needs_layout_passes=False),
    )(_body)
    return k(inp, weights)


def kernel(input, weights):
    out = _run(input, weights)
    return lax.stop_gradient(out.reshape(_B, _S, _D))


# speedup vs baseline: 1.9254x; 1.0025x over previous
import functools
import jax
import jax.numpy as jnp
from jax import lax
from jax.experimental import pallas as pl
from jax.experimental.pallas import tpu as pltpu, tpu_sc as plsc

_B, _S = 4, 4096
_D = 1024
_L = 16
_NW = 32
_ROWS_PER_W = (_B * _S) // _NW
_CHUNKS_PER_ROW = _S // _ROWS_PER_W
_GCHUNK = 16
_NG = _ROWS_PER_W // _GCHUNK
_DEPTH = 6
_OG = 3


def _body(inp_hbm, table_hbm, out_hbm, inp_v, idx_v, *bufs_and_sems):
    rows = bufs_and_sems[:_DEPTH]
    gsem = bufs_and_sems[_DEPTH : _DEPTH + _OG]
    wsem = bufs_and_sems[_DEPTH + _OG :]
    nc = 2
    wid = lax.axis_index("s") * nc + lax.axis_index("c")
    b = wid // _CHUNKS_PER_ROW
    c = wid % _CHUNKS_PER_ROW

    pltpu.sync_copy(inp_hbm.at[b], inp_v)

    def count_step(j, acc):
        x = inp_v[pl.ds(j * _L, _L)]
        return acc + jnp.where(x != 0, 1, 0).astype(jnp.int32)

    nprefix_vecs = c * (_ROWS_PER_W // _L)
    acc = lax.fori_loop(0, nprefix_vecs, count_step, jnp.zeros((_L,), jnp.int32))
    carry0 = jnp.sum(acc)

    base_vec = nprefix_vecs

    def pos_step(j, carry):
        x = inp_v[pl.ds((base_vec + j) * _L, _L)]
        m = jnp.where(x != 0, 1, 0).astype(jnp.int32)
        pos = (carry + jnp.cumsum(m)) * m
        idx_v[pl.ds(j * _L, _L)] = pos
        return carry + jnp.sum(m)

    lax.fori_loop(0, _ROWS_PER_W // _L, pos_step, carry0)

    out_base = wid * _ROWS_PER_W

    def gather(g):
        return pltpu.async_copy(
            table_hbm.at[idx_v.at[pl.ds(g * _GCHUNK, _GCHUNK)]],
            rows[g % _DEPTH],
            gsem[g % _OG],
        )

    writes = [None] * _NG
    gathers = [None] * _NG
    for g in range(_OG):
        gathers[g] = gather(g)
    for g in range(_NG):
        p = g % _DEPTH
        gathers[g].wait()
        if g + _OG < _NG:
            if g - (_DEPTH - _OG) >= 0:
                writes[g - (_DEPTH - _OG)].wait()
            gathers[g + _OG] = gather(g + _OG)
        writes[g] = pltpu.async_copy(
            rows[p], out_hbm.at[pl.ds(out_base + g * _GCHUNK, _GCHUNK)], wsem[p]
        )
    for g in range(max(0, _NG - _DEPTH), _NG):
        writes[g].wait()


@jax.jit
def _run(inp, weights):
    mesh = plsc.VectorSubcoreMesh(core_axis_name="c", subcore_axis_name="s")
    k = functools.partial(
        pl.kernel,
        mesh=mesh,
        out_type=jax.ShapeDtypeStruct((_B * _S, _D), jnp.float32),
        scratch_types=[
            pltpu.VMEM((_S,), jnp.int32),
            pltpu.VMEM((_ROWS_PER_W,), jnp.int32),
        ]
        + [pltpu.VMEM((_GCHUNK, _D), jnp.float32) for _ in range(_DEPTH)]
        + [pltpu.SemaphoreType.DMA for _ in range(_OG + _DEPTH)],
        compiler_params=pltpu.CompilerParams(needs_layout_passes=False),
    )(_body)
    return k(inp, weights)


def kernel(input, weights):
    out = _run(input, weights)
    return lax.stop_gradient(out.reshape(_B, _S, _D))
